# baseline (speedup 1.0000x reference)
import functools

import jax
import jax.numpy as jnp
from jax import lax
from jax.experimental import pallas as pl
from jax.experimental.pallas import tpu as pltpu
from jax.experimental.pallas import tpu_sc as plsc

N = 320000
D = 128
NUM_SEG = 10000

NC = 2
NS = 16
NW = NC * NS

CHUNK = 64
GRP = CHUNK
NCHUNKS = N // CHUNK
NOCT = NCHUNKS // 8
OCT_BASE = NOCT // NW
OCT_REM = NOCT - OCT_BASE * NW
NBUF = 2
STRIPE = 624
STRIPE_LAST = NUM_SEG - (NS - 1) * STRIPE


def _sc_segment_partials(x, batch2d, zeros_stripe):
    mesh = plsc.VectorSubcoreMesh(core_axis_name="c", subcore_axis_name="s")

    @functools.partial(
        pl.kernel,
        mesh=mesh,
        out_type=jax.ShapeDtypeStruct((NC, NUM_SEG, D), jnp.float32),
        scratch_types=[
            pltpu.VMEM((CHUNK, D), jnp.float32),
            pltpu.VMEM((CHUNK, D), jnp.float32),
            pltpu.VMEM(((OCT_BASE + 1) * 8, GRP), jnp.int32),
            pltpu.VMEM_SHARED((NUM_SEG, D), jnp.float32),
            pltpu.SemaphoreType.DMA,
            pltpu.SemaphoreType.DMA,
            pltpu.SemaphoreType.DMA,
        ],
    )
    def k(x_hbm, b_hbm, z_hbm, out_hbm, rows0, rows1, idx_all,
          acc, sem0, sem1, sem_sc):
        cid = lax.axis_index("c")
        sid = lax.axis_index("s")
        wid = sid * NC + cid

        @pl.when(sid < NS - 1)
        def _():
            pltpu.sync_copy(z_hbm.at[pl.ds(0, STRIPE)],
                            acc.at[pl.ds(sid * STRIPE, STRIPE)])

        @pl.when(sid == NS - 1)
        def _():
            pltpu.sync_copy(z_hbm,
                            acc.at[pl.ds((NS - 1) * STRIPE, STRIPE_LAST)])

        plsc.subcore_barrier()

        base_oct = wid * OCT_BASE + jnp.minimum(wid, OCT_REM)
        n_oct = OCT_BASE + (wid < OCT_REM).astype(jnp.int32)
        count = n_oct * 8
        base_chunk = base_oct * 8
        slots = ((rows0, sem0), (rows1, sem1))

        @pl.when(wid < OCT_REM)
        def _():
            pltpu.sync_copy(
                b_hbm.at[pl.ds(base_oct * 8, (OCT_BASE + 1) * 8)], idx_all)

        @pl.when(wid >= OCT_REM)
        def _():
            pltpu.sync_copy(
                b_hbm.at[pl.ds(base_oct * 8, OCT_BASE * 8)],
                idx_all.at[pl.ds(0, OCT_BASE * 8)])

        def fill(b, i):
            rows_v, sem = slots[b]
            c = base_chunk + i
            pltpu.async_copy(x_hbm.at[pl.ds(c * CHUNK, CHUNK)], rows_v, sem)

        def wait_fill(b, i):
            rows_v, sem = slots[b]
            c = base_chunk + i
            pltpu.make_async_copy(
                x_hbm.at[pl.ds(c * CHUNK, CHUNK)], rows_v, sem).wait()

        def fire_scatter(b, i):
            rows_v, _ = slots[b]
            pltpu.async_copy(rows_v, acc.at[idx_all.at[i]], sem_sc, add=True)

        def wait_scatter(b, i):
            rows_v, _ = slots[b]
            pltpu.make_async_copy(
                rows_v, acc.at[idx_all.at[i]], sem_sc).wait()

        for b in range(NBUF):
            fill(b, b)

        def body(p, carry):
            for k in range(8):
                i = p * 8 + k
                b = k % NBUF
                bp = (k - 1) % NBUF
                wait_fill(b, i)
                fire_scatter(b, i)

                def retire():
                    wait_scatter(bp, i - 1)

                    @pl.when(i - 1 + NBUF < count)
                    def _():
                        fill(bp, i - 1 + NBUF)

                if k == 0:
                    @pl.when(i > 0)
                    def _():
                        retire()
                else:
                    retire()

            return carry

        lax.fori_loop(0, count // 8, body, 0)
        wait_scatter((8 - 1) % NBUF, count - 1)
        plsc.subcore_barrier()

        @pl.when(sid < NS - 1)
        def _():
            pltpu.sync_copy(
                acc.at[pl.ds(sid * STRIPE, STRIPE)],
                out_hbm.at[cid].at[pl.ds(sid * STRIPE, STRIPE)],
            )

        @pl.when(sid == NS - 1)
        def _():
            pltpu.sync_copy(
                acc.at[pl.ds((NS - 1) * STRIPE, STRIPE_LAST)],
                out_hbm.at[cid].at[pl.ds((NS - 1) * STRIPE, STRIPE_LAST)],
            )

    return k(x, batch2d, zeros_stripe)


def _add_partials(partials):
    def body(a_ref, b_ref, o_ref):
        o_ref[...] = a_ref[0] + b_ref[0]

    blk = 1000
    return pl.pallas_call(
        body,
        grid=(NUM_SEG // blk,),
        in_specs=[
            pl.BlockSpec((1, blk, D), lambda i: (0, i, 0)),
            pl.BlockSpec((1, blk, D), lambda i: (1, i, 0)),
        ],
        out_specs=pl.BlockSpec((blk, D), lambda i: (i, 0)),
        out_shape=jax.ShapeDtypeStruct((NUM_SEG, D), jnp.float32),
    )(partials, partials)


@jax.jit
def kernel(x, batch):
    batch2d = batch.astype(jnp.int32).reshape(N // GRP, GRP)
    zeros_stripe = jnp.zeros((STRIPE_LAST, D), jnp.float32)
    partials = _sc_segment_partials(x, batch2d, zeros_stripe)
    return _add_partials(partials)


# speedup vs baseline: 1.3875x; 1.3875x over previous
import functools

import jax
import jax.numpy as jnp
from jax import lax
from jax.experimental import pallas as pl
from jax.experimental.pallas import tpu as pltpu
from jax.experimental.pallas import tpu_sc as plsc

N = 320000
D = 128
NUM_SEG = 10000

NC = 2
NS = 16
NW = NC * NS

CHUNK = 128
GRP = 16
GRPS_PER_CHUNK = CHUNK // GRP
NCHUNKS = N // CHUNK
STRIPE = 624
STRIPE_LAST = NUM_SEG - (NS - 1) * STRIPE


def _sc_segment_partials(x, batch2d, zeros_stripe):
    mesh = plsc.VectorSubcoreMesh(core_axis_name="c", subcore_axis_name="s")

    @functools.partial(
        pl.kernel,
        mesh=mesh,
        out_type=jax.ShapeDtypeStruct((NC, NUM_SEG, D), jnp.float32),
        scratch_types=[
            pltpu.VMEM((CHUNK, D), jnp.float32),
            pltpu.VMEM((CHUNK, D), jnp.float32),
            pltpu.VMEM((GRPS_PER_CHUNK, GRP), jnp.int32),
            pltpu.VMEM((GRPS_PER_CHUNK, GRP), jnp.int32),
            pltpu.VMEM_SHARED((NUM_SEG, D), jnp.float32),
            pltpu.SemaphoreType.DMA,
            pltpu.SemaphoreType.DMA,
            pltpu.SemaphoreType.DMA,
        ],
    )
    def k(x_hbm, b_hbm, z_hbm, out_hbm, rows0, rows1, idx0, idx1, acc,
          sem0, sem1, sem_sc):
        cid = lax.axis_index("c")
        sid = lax.axis_index("s")
        wid = sid * NC + cid

        @pl.when(sid < NS - 1)
        def _():
            pltpu.sync_copy(z_hbm.at[pl.ds(0, STRIPE)],
                            acc.at[pl.ds(sid * STRIPE, STRIPE)])

        @pl.when(sid == NS - 1)
        def _():
            pltpu.sync_copy(z_hbm,
                            acc.at[pl.ds((NS - 1) * STRIPE, STRIPE_LAST)])

        plsc.subcore_barrier()

        nchunks_w = (NCHUNKS - wid + NW - 1) // NW
        slots = ((rows0, idx0, sem0), (rows1, idx1, sem1))

        def fill(slot, i):
            rows_v, idx_v, sem = slot
            c = wid + i * NW
            pltpu.async_copy(x_hbm.at[pl.ds(c * CHUNK, CHUNK)], rows_v, sem)
            pltpu.async_copy(
                b_hbm.at[pl.ds(c * GRPS_PER_CHUNK, GRPS_PER_CHUNK)], idx_v, sem)

        def wait_fill(slot, i):
            rows_v, idx_v, sem = slot
            c = wid + i * NW
            pltpu.make_async_copy(
                x_hbm.at[pl.ds(c * CHUNK, CHUNK)], rows_v, sem).wait()
            pltpu.make_async_copy(
                b_hbm.at[pl.ds(c * GRPS_PER_CHUNK, GRPS_PER_CHUNK)], idx_v,
                sem).wait()

        def scatter(slot):
            rows_v, idx_v, _ = slot
            hs = [
                pltpu.async_copy(
                    rows_v.at[pl.ds(j * GRP, GRP)],
                    acc.at[idx_v.at[j]],
                    sem_sc,
                    add=True,
                )
                for j in range(GRPS_PER_CHUNK)
            ]
            for h in hs:
                h.wait()

        fill(slots[0], 0)
        fill(slots[1], 1)

        def body(p, carry):
            for b in (0, 1):
                i = 2 * p + b

                @pl.when(i < nchunks_w)
                def _():
                    wait_fill(slots[b], i)
                    scatter(slots[b])

                    @pl.when(i + 2 < nchunks_w)
                    def _():
                        fill(slots[b], i + 2)

            return carry

        lax.fori_loop(0, (nchunks_w + 1) // 2, body, 0)
        plsc.subcore_barrier()

        @pl.when(sid < NS - 1)
        def _():
            pltpu.sync_copy(
                acc.at[pl.ds(sid * STRIPE, STRIPE)],
                out_hbm.at[cid].at[pl.ds(sid * STRIPE, STRIPE)],
            )

        @pl.when(sid == NS - 1)
        def _():
            pltpu.sync_copy(
                acc.at[pl.ds((NS - 1) * STRIPE, STRIPE_LAST)],
                out_hbm.at[cid].at[pl.ds((NS - 1) * STRIPE, STRIPE_LAST)],
            )

    return k(x, batch2d, zeros_stripe)


def _add_partials(partials):
    def body(a_ref, b_ref, o_ref):
        o_ref[...] = a_ref[0] + b_ref[0]

    blk = 1000
    return pl.pallas_call(
        body,
        grid=(NUM_SEG // blk,),
        in_specs=[
            pl.BlockSpec((1, blk, D), lambda i: (0, i, 0)),
            pl.BlockSpec((1, blk, D), lambda i: (1, i, 0)),
        ],
        out_specs=pl.BlockSpec((blk, D), lambda i: (i, 0)),
        out_shape=jax.ShapeDtypeStruct((NUM_SEG, D), jnp.float32),
    )(partials, partials)


@jax.jit
def kernel(x, batch):
    batch2d = batch.astype(jnp.int32).reshape(N // GRP, GRP)
    zeros_stripe = jnp.zeros((STRIPE_LAST, D), jnp.float32)
    partials = _sc_segment_partials(x, batch2d, zeros_stripe)
    return _add_partials(partials)
